# Initial kernel scaffold; baseline (speedup 1.0000x reference)
import jax
import jax.numpy as jnp
from jax.experimental import pallas as pl


def kernel(inputs, hidden_states, edge_index, edge_attr, W1, b1, g1, be1, W2, b2, g2, be2, Wl, bl, Wr, br, We, att, Wres, gb, Wih, bih, Whh, bhh, g3, be3, Wq, bq):
    raise NotImplementedError("write your pallas kernel here")


# speedup vs baseline: 6.9900x; 6.9900x over previous
import jax
import jax.numpy as jnp
from jax import lax
from jax.experimental import pallas as pl
from jax.experimental.pallas import tpu as pltpu
from jax.experimental.pallas import tpu_sc as plsc

N = 10000
E = 320000
D_IN = 128
H = 64
D = 128
N_ACT = 14

NP = 10240
ROW_BLOCK = 1280
NB = NP // ROW_BLOCK

NC = 2
NS = 16
NW = NC * NS
CHUNK = 80
EPW = E // NW
ITERS = EPW // CHUNK
RPT = NP // NS
DK = D // 16
G = CHUNK // 16


def _ln(x, g, b):
    m = jnp.mean(x, axis=-1, keepdims=True)
    v = jnp.mean((x - m) ** 2, axis=-1, keepdims=True)
    return (x - m) * lax.rsqrt(v + 1e-5) * g + b


def _pre_body(in_ref, w1, b1, g1, be1, w2, b2, g2, be2, wl, bl, wr, br, wres,
              xl_ref, xr_ref, xres_ref):
    x = in_ref[...]
    x = jnp.maximum(jnp.dot(x, w1[...], preferred_element_type=jnp.float32)
                    + b1[...], 0.0)
    x = _ln(x, g1[...], be1[...])
    x = jnp.maximum(jnp.dot(x, w2[...], preferred_element_type=jnp.float32)
                    + b2[...], 0.0)
    x = _ln(x, g2[...], be2[...])
    xl_ref[...] = jnp.dot(x, wl[...], preferred_element_type=jnp.float32) + bl[...]
    xr_ref[...] = jnp.dot(x, wr[...], preferred_element_type=jnp.float32) + br[...]
    xres_ref[...] = jnp.dot(x, wres[...], preferred_element_type=jnp.float32)


def _pre_tc(inputs, W1, b1, g1, be1, W2, b2, g2, be2, Wl, bl, Wr, br, Wres):
    full = lambda s: pl.BlockSpec(s, lambda i: (0, 0))
    row = lambda d: pl.BlockSpec((ROW_BLOCK, d), lambda i: (i, 0))
    return pl.pallas_call(
        _pre_body,
        grid=(NB,),
        in_specs=[row(D_IN), full((D_IN, H)), full((1, H)), full((1, H)),
                  full((1, H)), full((H, H)), full((1, H)), full((1, H)),
                  full((1, H)), full((H, D)), full((1, D)), full((H, D)),
                  full((1, D)), full((H, D))],
        out_specs=[row(D), row(D), row(D)],
        out_shape=[jax.ShapeDtypeStruct((NP, D), jnp.float32)] * 3,
    )(inputs, W1, b1, g1, be1, W2, b2, g2, be2, Wl, bl, Wr, br, Wres)


def _edge_body(xl_hbm, xr_hbm, src_hbm, dst_hbm, ea0_hbm, ea1_hbm, ea2_hbm,
               we_hbm, att_hbm,
               agg_out, den_out,
               agg_sh, xl_buf, xr_buf, src_buf, dst_buf, ea_buf,
               lg_buf, den_loc, att_v, we_v, sem1, sem2):
    cid = lax.axis_index("c")
    sid = lax.axis_index("s")
    wid = sid * NC + cid

    z16 = jnp.zeros((16,), jnp.float32)
    iota16 = jnp.arange(16, dtype=jnp.int32)
    lane15 = iota16 == 15

    def _zb(r, _):
        for k in range(DK):
            xl_buf[r, pl.ds(16 * k, 16)] = z16
        return 0
    lax.fori_loop(0, CHUNK, _zb, 0)

    def _zd(r, _):
        den_loc[pl.ds(16 * r, 16)] = z16
        return 0
    lax.fori_loop(0, NP // 16, _zd, 0)

    r0 = sid * RPT

    def _za(j, _):
        pltpu.sync_copy(xl_buf, agg_sh.at[pl.ds(r0 + j * CHUNK, CHUNK), :])
        return 0
    lax.fori_loop(0, RPT // CHUNK, _za, 0)

    pltpu.sync_copy(att_hbm, att_v)
    pltpu.sync_copy(we_hbm, we_v)

    plsc.subcore_barrier()

    def _iter(i, _):
        base = wid * EPW + i * CHUNK
        pltpu.sync_copy(src_hbm.at[pl.ds(base, CHUNK)], src_buf)
        pltpu.sync_copy(dst_hbm.at[pl.ds(base, CHUNK)], dst_buf)
        pltpu.sync_copy(ea0_hbm.at[pl.ds(base, CHUNK)], ea_buf.at[0])
        pltpu.sync_copy(ea1_hbm.at[pl.ds(base, CHUNK)], ea_buf.at[1])
        pltpu.sync_copy(ea2_hbm.at[pl.ds(base, CHUNK)], ea_buf.at[2])
        c1 = pltpu.async_copy(xl_hbm.at[src_buf], xl_buf, sem1)
        c2 = pltpu.async_copy(xr_hbm.at[dst_buf], xr_buf, sem2)
        c1.wait()
        c2.wait()

        attc = [att_v[pl.ds(16 * k, 16)] for k in range(DK)]
        wec = [[we_v[j, pl.ds(16 * k, 16)] for k in range(DK)] for j in range(3)]

        def _grp(g, _):
            a0v = ea_buf[0, pl.ds(16 * g, 16)]
            a1v = ea_buf[1, pl.ds(16 * g, 16)]
            a2v = ea_buf[2, pl.ds(16 * g, 16)]
            for e16 in range(16):
                e = 16 * g + e16
                a0b = jnp.full((16,), a0v[e16], jnp.float32)
                a1b = jnp.full((16,), a1v[e16], jnp.float32)
                a2b = jnp.full((16,), a2v[e16], jnp.float32)
                parts = []
                for k in range(DK):
                    m = (xl_buf[e, pl.ds(16 * k, 16)]
                         + xr_buf[e, pl.ds(16 * k, 16)]
                         + a0b * wec[0][k] + a1b * wec[1][k] + a2b * wec[2][k])
                    m = jnp.maximum(m, 0.2 * m)
                    parts.append(attc[k] * m)
                s01 = parts[0] + parts[1]
                s23 = parts[2] + parts[3]
                s45 = parts[4] + parts[5]
                s67 = parts[6] + parts[7]
                sc = jnp.cumsum((s01 + s23) + (s45 + s67))
                plsc.store_compressed(lg_buf.at[pl.ds(e, 16)], sc, mask=lane15)
            return 0

        lax.fori_loop(0, G, _grp, 0)

        def _scl(g, _):
            w = jnp.exp(lg_buf[pl.ds(16 * g, 16)])
            dstv = dst_buf[pl.ds(16 * g, 16)]
            plsc.addupdate_scatter(den_loc, [dstv], w)
            for e16 in range(16):
                ws = w[e16]
                e = 16 * g + e16
                for k in range(DK):
                    xl_buf[e, pl.ds(16 * k, 16)] = ws * xl_buf[e, pl.ds(16 * k, 16)]
            return 0

        lax.fori_loop(0, G, _scl, 0)

        pltpu.sync_copy(xl_buf, agg_sh.at[dst_buf], add=True)
        return 0

    lax.fori_loop(0, ITERS, _iter, 0)

    pltpu.sync_copy(den_loc, den_out.at[pl.ds(wid * NP, NP)])

    plsc.subcore_barrier()

    def _out(j, _):
        rr = r0 + j * CHUNK
        pltpu.sync_copy(agg_sh.at[pl.ds(rr, CHUNK), :], xl_buf)
        pltpu.sync_copy(xl_buf, agg_out.at[cid, pl.ds(rr, CHUNK), :])
        return 0
    lax.fori_loop(0, RPT // CHUNK, _out, 0)


def _edge_sc(xl, xr, src, dst, ea0, ea1, ea2, We, att):
    mesh = plsc.VectorSubcoreMesh(core_axis_name="c", subcore_axis_name="s")
    f32 = jnp.float32
    kern = pl.kernel(
        _edge_body,
        out_type=(jax.ShapeDtypeStruct((NC, NP, D), f32),
                  jax.ShapeDtypeStruct((NW * NP,), f32)),
        mesh=mesh,
        compiler_params=pltpu.CompilerParams(needs_layout_passes=False),
        scratch_types=[
            pltpu.VMEM_SHARED((NP, D), f32),
            pltpu.VMEM((CHUNK, D), f32),
            pltpu.VMEM((CHUNK, D), f32),
            pltpu.VMEM((CHUNK,), jnp.int32),
            pltpu.VMEM((CHUNK,), jnp.int32),
            pltpu.VMEM((3, CHUNK), f32),
            pltpu.VMEM((CHUNK + 16,), f32),
            pltpu.VMEM((NP,), f32),
            pltpu.VMEM((D,), f32),
            pltpu.VMEM((3, D), f32),
            pltpu.SemaphoreType.DMA,
            pltpu.SemaphoreType.DMA,
        ],
    )
    return kern(xl, xr, src, dst, ea0, ea1, ea2, We, att)


def _post_body(aggp, denp, xres, hin_ref, gb, wih, bih, whh, bhh, g3, be3,
               wq, bq, q_ref, h_ref):
    agg = aggp[0] + aggp[1]
    den = jnp.sum(denp[...], axis=0).reshape(ROW_BLOCK, 1)
    out = agg / (den + 1e-16) + xres[...] + gb[...]
    h = jnp.maximum(out, 0.0)
    h_in = hin_ref[...]
    gi = jnp.dot(h, wih[...], preferred_element_type=jnp.float32) + bih[...]
    gh = jnp.dot(h_in, whh[...], preferred_element_type=jnp.float32) + bhh[...]
    ir, iz, inn = gi[:, 0:D], gi[:, D:2 * D], gi[:, 2 * D:3 * D]
    hr, hz, hn = gh[:, 0:D], gh[:, D:2 * D], gh[:, 2 * D:3 * D]
    r = jax.nn.sigmoid(ir + hr)
    z = jax.nn.sigmoid(iz + hz)
    n = jnp.tanh(inn + r * hn)
    h_new = (1.0 - z) * n + z * h_in
    q_ref[...] = jnp.dot(_ln(h_new, g3[...], be3[...]), wq[...],
                         preferred_element_type=jnp.float32) + bq[...]
    h_ref[...] = h_new


def _post_tc(aggp, denp, xres, hidden, gb, Wih, bih, Whh, bhh, g3, be3,
             Wq_pad, bq_pad):
    full = lambda s: pl.BlockSpec(s, lambda i: (0,) * len(s))
    row = lambda d: pl.BlockSpec((ROW_BLOCK, d), lambda i: (i, 0))
    return pl.pallas_call(
        _post_body,
        grid=(NB,),
        in_specs=[pl.BlockSpec((NC, ROW_BLOCK, D), lambda i: (0, i, 0)),
                  pl.BlockSpec((NW, ROW_BLOCK), lambda i: (0, i)),
                  row(D), row(D), full((1, D)),
                  full((D, 3 * D)), full((1, 3 * D)),
                  full((D, 3 * D)), full((1, 3 * D)),
                  full((1, D)), full((1, D)),
                  full((D, 128)), full((1, 128))],
        out_specs=[row(128), row(D)],
        out_shape=[jax.ShapeDtypeStruct((NP, 128), jnp.float32),
                   jax.ShapeDtypeStruct((NP, D), jnp.float32)],
    )(aggp, denp, xres, hidden, gb, Wih, bih, Whh, bhh, g3, be3, Wq_pad, bq_pad)


def kernel(inputs, hidden_states, edge_index, edge_attr, W1, b1, g1, be1,
           W2, b2, g2, be2, Wl, bl, Wr, br, We, att, Wres, gb, Wih, bih,
           Whh, bhh, g3, be3, Wq, bq):
    r1 = lambda v: v.reshape(1, -1)
    inputs_p = jnp.pad(inputs, ((0, NP - N), (0, 0)))
    hidden_p = jnp.pad(hidden_states, ((0, NP - N), (0, 0)))
    xl, xr, xres = _pre_tc(inputs_p, W1, r1(b1), r1(g1), r1(be1), W2, r1(b2),
                           r1(g2), r1(be2), Wl, r1(bl), Wr, r1(br), Wres)
    src = edge_index[0]
    dst = edge_index[1]
    aggp, den_flat = _edge_sc(xl, xr, src, dst, edge_attr[:, 0],
                              edge_attr[:, 1], edge_attr[:, 2], We, att)
    denp = den_flat.reshape(NW, NP)
    Wq_pad = jnp.zeros((D, 128), jnp.float32).at[:, :N_ACT].set(Wq)
    bq_pad = jnp.zeros((1, 128), jnp.float32).at[:, :N_ACT].set(bq.reshape(1, -1))
    q_full, h_new = _post_tc(aggp, denp, xres, hidden_p, r1(gb), Wih,
                             r1(bih), Whh, r1(bhh), r1(g3), r1(be3),
                             Wq_pad, bq_pad)
    return (q_full[:N, :N_ACT], h_new[:N, :])
